# Initial kernel scaffold; baseline (speedup 1.0000x reference)
import jax
import jax.numpy as jnp
from jax.experimental import pallas as pl


def kernel(h, edge_index, edge_attr, valences, W1, b1, W2, b2, U1, bu1, U2, bu2, V1, bv1, V2, bv2):
    raise NotImplementedError("write your pallas kernel here")


# speedup vs baseline: 15.0980x; 15.0980x over previous
import functools

import jax
import jax.numpy as jnp
from jax import lax
from jax.experimental import pallas as pl
from jax.experimental.pallas import tpu as pltpu
from jax.experimental.pallas import tpu_sc as plsc

N = 10000
E = 320000
D = 128

NC = 2
NS = 16
NW = NC * NS
L = 16

NPAD = 10240
ROWS_PER_TILE = NPAD // NS
CH = 80
CPT = 128
EPT = CH * CPT
EPAD = EPT * NW
KDIM = 136


def _prologue_body(x_ref, wa_ref, wb_ref, b1_ref, a_ref, b_ref):
    x = x_ref[...]
    a_ref[...] = jnp.dot(x, wa_ref[...], preferred_element_type=jnp.float32) + b1_ref[...]
    b_ref[...] = jnp.dot(x, wb_ref[...], preferred_element_type=jnp.float32)


def _tc_prologue(x, wa, wb, b1):
    blk = 1024
    grid = NPAD // blk
    return pl.pallas_call(
        _prologue_body,
        grid=(grid,),
        in_specs=[
            pl.BlockSpec((blk, KDIM), lambda i: (i, 0)),
            pl.BlockSpec((KDIM, D), lambda i: (0, 0)),
            pl.BlockSpec((KDIM, D), lambda i: (0, 0)),
            pl.BlockSpec((1, D), lambda i: (0, 0)),
        ],
        out_specs=[
            pl.BlockSpec((blk, D), lambda i: (i, 0)),
            pl.BlockSpec((blk, D), lambda i: (i, 0)),
        ],
        out_shape=[
            jax.ShapeDtypeStruct((NPAD, D), jnp.float32),
            jax.ShapeDtypeStruct((NPAD, D), jnp.float32),
        ],
    )(x, wa, wb, b1)


def _epilogue_body(h_ref, sp_ref, t_ref, u1a_ref, w2_ref, u1b_ref, u2_ref,
                   b2_ref, bu1_ref, bu2_ref, o_ref):
    hb = h_ref[...]
    s = sp_ref[0] + sp_ref[1]
    w2u = jnp.dot(w2_ref[...], u1b_ref[...], preferred_element_type=jnp.float32)
    b2u = jnp.dot(b2_ref[...], u1b_ref[...], preferred_element_type=jnp.float32)
    ones = jnp.ones((NW, 1), jnp.float32)
    tv = lax.dot_general(t_ref[...], ones, (((0,), (0,)), ((), ())),
                         preferred_element_type=jnp.float32)
    z = (jnp.dot(hb, u1a_ref[...], preferred_element_type=jnp.float32)
         + jnp.dot(s, w2u, preferred_element_type=jnp.float32)
         + tv * b2u
         + bu1_ref[...])
    u = jnp.maximum(z, 0.0)
    o_ref[...] = hb + jnp.dot(u, u2_ref[...], preferred_element_type=jnp.float32) + bu2_ref[...]


def _tc_epilogue(h_pad, sp, t2, u1a, w2, u1b, u2, b2, bu1, bu2):
    blk = 1024
    grid = NPAD // blk
    return pl.pallas_call(
        _epilogue_body,
        grid=(grid,),
        in_specs=[
            pl.BlockSpec((blk, D), lambda i: (i, 0)),
            pl.BlockSpec((2, blk, D), lambda i: (0, i, 0)),
            pl.BlockSpec((NW, blk), lambda i: (0, i)),
            pl.BlockSpec((D, D), lambda i: (0, 0)),
            pl.BlockSpec((D, D), lambda i: (0, 0)),
            pl.BlockSpec((D, D), lambda i: (0, 0)),
            pl.BlockSpec((D, D), lambda i: (0, 0)),
            pl.BlockSpec((1, D), lambda i: (0, 0)),
            pl.BlockSpec((1, D), lambda i: (0, 0)),
            pl.BlockSpec((1, D), lambda i: (0, 0)),
        ],
        out_specs=pl.BlockSpec((blk, D), lambda i: (i, 0)),
        out_shape=jax.ShapeDtypeStruct((NPAD, D), jnp.float32),
    )(h_pad, sp, t2, u1a, w2, u1b, u2, b2, bu1, bu2)


def _sc_edge_body(a_hbm, b_hbm, rcg_hbm, vp_hbm, att_hbm,
                  sp_out, t_out,
                  s_sh, rcp, abuf, attb, rowu, colu, vp_loc, att_v,
                  t_loc, sem_a, sem_b):
    cid = lax.axis_index("c")
    sid = lax.axis_index("s")
    wid = sid * NC + cid

    pltpu.sync_copy(vp_hbm, vp_loc)
    pltpu.sync_copy(att_hbm, att_v)
    pltpu.sync_copy(rcg_hbm.at[pl.ds(wid * CPT, CPT // 2)], rcp)

    zeros16 = jnp.zeros((L,), jnp.float32)
    iota16 = lax.iota(jnp.int32, L)

    def _zero_row(r, _):
        for j in range(D // L):
            abuf[r, pl.ds(j * L, L)] = zeros16
        return 0

    lax.fori_loop(0, CH, _zero_row, 0)

    def _zero_t(r, _):
        t_loc[pl.ds(r * L, L)] = zeros16
        return 0

    lax.fori_loop(0, NPAD // L, _zero_t, 0)

    base = sid * ROWS_PER_TILE
    for p in range(ROWS_PER_TILE // CH):
        pltpu.sync_copy(abuf, s_sh.at[pl.ds(base + p * CH, CH)])
    plsc.subcore_barrier()

    def _chunk(k, _):
        for g in range(CH // L):
            sl = pl.ds(g * L, L)
            rc16 = rcp[k, sl]
            r16 = jnp.bitwise_and(rc16, jnp.int32(0xFFFF))
            c16 = jnp.right_shift(rc16, jnp.int32(16))
            rowu[sl] = r16
            colu[sl] = c16
            vr = jnp.bitwise_and(
                jnp.right_shift(plsc.load_gather(vp_loc, [jnp.right_shift(r16, jnp.int32(2))]),
                                jnp.left_shift(jnp.bitwise_and(r16, jnp.int32(3)), jnp.int32(3))),
                jnp.int32(7))
            vc = jnp.bitwise_and(
                jnp.right_shift(plsc.load_gather(vp_loc, [jnp.right_shift(c16, jnp.int32(2))]),
                                jnp.left_shift(jnp.bitwise_and(c16, jnp.int32(3)), jnp.int32(3))),
                jnp.int32(7))
            iv = vr * 8 + vc
            a16 = plsc.load_gather(att_v, [iv])
            attb[sl] = a16
            for l in range(L):
                plsc.addupdate_scatter(t_loc, [r16], a16, mask=iota16 == l)

        pltpu.async_copy(a_hbm.at[rowu], abuf, sem_a).wait()
        pltpu.async_copy(b_hbm.at[colu], abuf, sem_b, add=True).wait()

        def _group(g, _):
            att16 = attb[pl.ds(g * L, L)]
            for l in range(L):
                ae = att16[l]
                e = g * L + l
                for j in range(D // L):
                    sl = pl.ds(j * L, L)
                    abuf[e, sl] = jnp.maximum(abuf[e, sl], 0.0) * ae
            return 0

        lax.fori_loop(0, CH // L, _group, 0)
        pltpu.sync_copy(abuf, s_sh.at[rowu], add=True)
        return 0

    lax.fori_loop(0, CPT // 2, _chunk, 0)
    pltpu.sync_copy(rcg_hbm.at[pl.ds(wid * CPT + CPT // 2, CPT // 2)], rcp)
    lax.fori_loop(0, CPT // 2, _chunk, 0)
    plsc.subcore_barrier()

    pltpu.sync_copy(s_sh.at[pl.ds(base, ROWS_PER_TILE)],
                    sp_out.at[cid, pl.ds(base, ROWS_PER_TILE)])
    pltpu.sync_copy(t_loc, t_out.at[cid, sid])


def _sc_edge(a_pad, b_pad, rcg, vpack, att_tab):
    mesh = plsc.VectorSubcoreMesh(core_axis_name="c", subcore_axis_name="s",
                                  num_cores=NC, num_subcores=NS)
    f = pl.kernel(
        _sc_edge_body,
        out_type=[
            jax.ShapeDtypeStruct((NC, NPAD, D), jnp.float32),
            jax.ShapeDtypeStruct((NC, NS, NPAD), jnp.float32),
        ],
        mesh=mesh,
        compiler_params=pltpu.CompilerParams(needs_layout_passes=False),
        scratch_types=[
            pltpu.VMEM_SHARED((NPAD, D), jnp.float32),
            pltpu.VMEM((CPT // 2, CH), jnp.int32),
            pltpu.VMEM((CH, D), jnp.float32),
            pltpu.VMEM((CH,), jnp.float32),
            pltpu.VMEM((CH,), jnp.int32),
            pltpu.VMEM((CH,), jnp.int32),
            pltpu.VMEM((NPAD // 4,), jnp.int32),
            pltpu.VMEM((64,), jnp.float32),
            pltpu.VMEM((NPAD,), jnp.float32),
            pltpu.SemaphoreType.DMA,
            pltpu.SemaphoreType.DMA,
        ],
    )
    return f(a_pad, b_pad, rcg, vpack, att_tab)


def kernel(h, edge_index, edge_attr, valences, W1, b1, W2, b2,
           U1, bu1, U2, bu2, V1, bv1, V2, bv2):
    vf = valences.astype(jnp.float32)

    x = jnp.zeros((NPAD, KDIM), jnp.float32)
    x = x.at[:N, :D].set(h).at[:N, D].set(vf)
    wa = jnp.zeros((KDIM, D), jnp.float32).at[:D].set(W1[:D]).at[D].set(W1[2 * D])
    wb = jnp.zeros((KDIM, D), jnp.float32).at[:D].set(W1[D:2 * D]).at[D].set(W1[2 * D + 1])

    a_pad, b_pad = _tc_prologue(x, wa, wb, b1[None, :])

    vr_g = jnp.arange(5, dtype=jnp.float32)[:, None] * jnp.ones((1, 5), jnp.float32)
    vc_g = jnp.ones((5, 1), jnp.float32) * jnp.arange(5, dtype=jnp.float32)[None, :]
    pairs = jnp.stack([vr_g.ravel(), vc_g.ravel()], axis=-1)
    tab = jax.nn.sigmoid(jax.nn.relu(pairs @ V1 + bv1) @ V2 + bv2)[:, 0]
    att_tab = jnp.zeros((64,), jnp.float32).at[
        (jnp.arange(25) // 5) * 8 + jnp.arange(25) % 5].set(tab)

    pad_n = EPAD - E
    pad_rows = N + (jnp.arange(pad_n, dtype=jnp.int32) % (NPAD - N))
    row_all = jnp.concatenate([edge_index[0], pad_rows])
    col_all = jnp.concatenate([edge_index[1], pad_rows])
    rcg = (row_all | (col_all << 16)).reshape(NW * CPT, CH)
    v_pad = jnp.zeros((NPAD,), jnp.int32).at[:N].set(valences)
    v4 = v_pad.reshape(NPAD // 4, 4)
    vpack = (v4[:, 0] | (v4[:, 1] << 8) | (v4[:, 2] << 16) | (v4[:, 3] << 24))

    sp, t_out = _sc_edge(a_pad, b_pad, rcg, vpack, att_tab)
    t2 = t_out.reshape(NC * NS, NPAD)

    h_pad = jnp.zeros((NPAD, D), jnp.float32).at[:N].set(h)
    out = _tc_epilogue(h_pad, sp, t2, U1[:D], W2, U1[D:], U2,
                       b2[None, :], bu1[None, :], bu2[None, :])
    return out[:N]


# speedup vs baseline: 18.4117x; 1.2195x over previous
import functools

import jax
import jax.numpy as jnp
from jax import lax
from jax.experimental import pallas as pl
from jax.experimental.pallas import tpu as pltpu
from jax.experimental.pallas import tpu_sc as plsc

N = 10000
E = 320000
D = 128

NC = 2
NS = 16
NW = NC * NS
L = 16

NPAD = 10240
ROWS_PER_TILE = NPAD // NS
CH = 96
CPT = 106
EPT = CH * CPT
EPAD = EPT * NW
KDIM = 136


def _prologue_body(x_ref, wa_ref, wb_ref, b1_ref, a_ref, b_ref):
    x = x_ref[...]
    a_ref[...] = jnp.dot(x, wa_ref[...], preferred_element_type=jnp.float32) + b1_ref[...]
    b_ref[...] = jnp.dot(x, wb_ref[...], preferred_element_type=jnp.float32)


def _tc_prologue(x, wa, wb, b1):
    blk = 1024
    grid = NPAD // blk
    return pl.pallas_call(
        _prologue_body,
        grid=(grid,),
        in_specs=[
            pl.BlockSpec((blk, KDIM), lambda i: (i, 0)),
            pl.BlockSpec((KDIM, D), lambda i: (0, 0)),
            pl.BlockSpec((KDIM, D), lambda i: (0, 0)),
            pl.BlockSpec((1, D), lambda i: (0, 0)),
        ],
        out_specs=[
            pl.BlockSpec((blk, D), lambda i: (i, 0)),
            pl.BlockSpec((blk, D), lambda i: (i, 0)),
        ],
        out_shape=[
            jax.ShapeDtypeStruct((NPAD, D), jnp.float32),
            jax.ShapeDtypeStruct((NPAD, D), jnp.float32),
        ],
    )(x, wa, wb, b1)


def _epilogue_body(h_ref, sp_ref, t_ref, u1a_ref, w2_ref, u1b_ref, u2_ref,
                   b2_ref, bu1_ref, bu2_ref, o_ref):
    hb = h_ref[...]
    s = sp_ref[0] + sp_ref[1]
    w2u = jnp.dot(w2_ref[...], u1b_ref[...], preferred_element_type=jnp.float32)
    b2u = jnp.dot(b2_ref[...], u1b_ref[...], preferred_element_type=jnp.float32)
    ones = jnp.ones((NW, 1), jnp.float32)
    tv = lax.dot_general(t_ref[...], ones, (((0,), (0,)), ((), ())),
                         preferred_element_type=jnp.float32)
    z = (jnp.dot(hb, u1a_ref[...], preferred_element_type=jnp.float32)
         + jnp.dot(s, w2u, preferred_element_type=jnp.float32)
         + tv * b2u
         + bu1_ref[...])
    u = jnp.maximum(z, 0.0)
    o_ref[...] = hb + jnp.dot(u, u2_ref[...], preferred_element_type=jnp.float32) + bu2_ref[...]


def _tc_epilogue(h_pad, sp, t2, u1a, w2, u1b, u2, b2, bu1, bu2):
    blk = 1024
    grid = NPAD // blk
    return pl.pallas_call(
        _epilogue_body,
        grid=(grid,),
        in_specs=[
            pl.BlockSpec((blk, D), lambda i: (i, 0)),
            pl.BlockSpec((2, blk, D), lambda i: (0, i, 0)),
            pl.BlockSpec((NW, blk), lambda i: (0, i)),
            pl.BlockSpec((D, D), lambda i: (0, 0)),
            pl.BlockSpec((D, D), lambda i: (0, 0)),
            pl.BlockSpec((D, D), lambda i: (0, 0)),
            pl.BlockSpec((D, D), lambda i: (0, 0)),
            pl.BlockSpec((1, D), lambda i: (0, 0)),
            pl.BlockSpec((1, D), lambda i: (0, 0)),
            pl.BlockSpec((1, D), lambda i: (0, 0)),
        ],
        out_specs=pl.BlockSpec((blk, D), lambda i: (i, 0)),
        out_shape=jax.ShapeDtypeStruct((NPAD, D), jnp.float32),
    )(h_pad, sp, t2, u1a, w2, u1b, u2, b2, bu1, bu2)


def _sc_edge_body(a_hbm, b_hbm, rcg_hbm, vp_hbm, att_hbm,
                  sp_out, t_out,
                  s_sh, rci, abuf, attb, rowu, colu, vp_loc, att_v,
                  t_loc, sem_a, sem_b, sem_i0, sem_i1, sem_s0, sem_s1):
    cid = lax.axis_index("c")
    sid = lax.axis_index("s")
    wid = sid * NC + cid
    crow = wid * CPT

    pltpu.sync_copy(vp_hbm, vp_loc)
    pltpu.sync_copy(att_hbm, att_v)

    zeros16 = jnp.zeros((L,), jnp.float32)
    iota16 = lax.iota(jnp.int32, L)
    sem_i = (sem_i0, sem_i1)
    sem_s = (sem_s0, sem_s1)

    def _zero_row(r, _):
        for b in range(2):
            for j in range(D // L):
                abuf[b, r, pl.ds(j * L, L)] = zeros16
        return 0

    lax.fori_loop(0, CH, _zero_row, 0)

    def _zero_t(r, _):
        t_loc[pl.ds(r * L, L)] = zeros16
        return 0

    lax.fori_loop(0, NPAD // L, _zero_t, 0)

    base = sid * ROWS_PER_TILE
    for p in range(ROWS_PER_TILE // CH):
        pltpu.sync_copy(abuf.at[0], s_sh.at[pl.ds(base + p * CH, CH)])
    for p in range(ROWS_PER_TILE % CH // L):
        pltpu.sync_copy(abuf.at[0, pl.ds(0, L)],
                        s_sh.at[pl.ds(base + (ROWS_PER_TILE // CH) * CH + p * L, L)])
    plsc.subcore_barrier()

    for b in range(2):
        for g in range(CH // L):
            rowu[b, pl.ds(g * L, L)] = iota16 + (NPAD - L)
        pltpu.async_copy(abuf.at[b], s_sh.at[rowu.at[b]], sem_s[b], add=True)
        pltpu.async_copy(rcg_hbm.at[pl.ds(crow + b, 1)], rci.at[pl.ds(b, 1)],
                         sem_i[b])

    def _do_chunk(k, b):
        pltpu.make_async_copy(abuf.at[b], s_sh.at[rowu.at[b]], sem_s[b]).wait()
        pltpu.make_async_copy(rcg_hbm.at[pl.ds(0, 1)], rci.at[pl.ds(b, 1)],
                              sem_i[b]).wait()
        for g in range(CH // L):
            sl = pl.ds(g * L, L)
            rc16 = rci[b, sl]
            rowu[b, sl] = jnp.bitwise_and(rc16, jnp.int32(0xFFFF))
            colu[b, sl] = jnp.right_shift(rc16, jnp.int32(16))
        pltpu.async_copy(rcg_hbm.at[pl.ds(crow + k + 2, 1)],
                         rci.at[pl.ds(b, 1)], sem_i[b])
        ca = pltpu.async_copy(a_hbm.at[rowu.at[b]], abuf.at[b], sem_a)
        for g in range(CH // L):
            sl = pl.ds(g * L, L)
            r16 = rowu[b, sl]
            c16 = colu[b, sl]
            vr = jnp.bitwise_and(
                jnp.right_shift(plsc.load_gather(vp_loc, [jnp.right_shift(r16, jnp.int32(3))]),
                                jnp.left_shift(jnp.bitwise_and(r16, jnp.int32(7)), jnp.int32(2))),
                jnp.int32(15))
            vc = jnp.bitwise_and(
                jnp.right_shift(plsc.load_gather(vp_loc, [jnp.right_shift(c16, jnp.int32(3))]),
                                jnp.left_shift(jnp.bitwise_and(c16, jnp.int32(7)), jnp.int32(2))),
                jnp.int32(15))
            iv = vr * 8 + vc
            a16 = plsc.load_gather(att_v, [iv])
            attb[sl] = a16
            for l in range(L):
                plsc.addupdate_scatter(t_loc, [r16], a16, mask=iota16 == l)
        ca.wait()
        pltpu.async_copy(b_hbm.at[colu.at[b]], abuf.at[b], sem_b,
                         add=True).wait()

        def _group(g, _):
            att16 = attb[pl.ds(g * L, L)]
            for l in range(L):
                ae = att16[l]
                e = g * L + l
                for j in range(D // L):
                    sl = pl.ds(j * L, L)
                    abuf[b, e, sl] = jnp.maximum(abuf[b, e, sl], 0.0) * ae
            return 0

        lax.fori_loop(0, CH // L, _group, 0)
        pltpu.async_copy(abuf.at[b], s_sh.at[rowu.at[b]], sem_s[b], add=True)

    def _pair(i, _):
        _do_chunk(i * 2, 0)
        _do_chunk(i * 2 + 1, 1)
        return 0

    lax.fori_loop(0, CPT // 2, _pair, 0)

    for b in range(2):
        pltpu.make_async_copy(abuf.at[b], s_sh.at[rowu.at[b]], sem_s[b]).wait()
        pltpu.make_async_copy(rcg_hbm.at[pl.ds(0, 1)], rci.at[pl.ds(b, 1)],
                              sem_i[b]).wait()
    plsc.subcore_barrier()

    pltpu.sync_copy(s_sh.at[pl.ds(base, ROWS_PER_TILE)],
                    sp_out.at[cid, pl.ds(base, ROWS_PER_TILE)])
    pltpu.sync_copy(t_loc, t_out.at[cid, sid])


def _sc_edge(a_pad, b_pad, rcg, vpack, att_tab):
    mesh = plsc.VectorSubcoreMesh(core_axis_name="c", subcore_axis_name="s",
                                  num_cores=NC, num_subcores=NS)
    f = pl.kernel(
        _sc_edge_body,
        out_type=[
            jax.ShapeDtypeStruct((NC, NPAD, D), jnp.float32),
            jax.ShapeDtypeStruct((NC, NS, NPAD), jnp.float32),
        ],
        mesh=mesh,
        compiler_params=pltpu.CompilerParams(needs_layout_passes=False),
        scratch_types=[
            pltpu.VMEM_SHARED((NPAD, D), jnp.float32),
            pltpu.VMEM((2, CH), jnp.int32),
            pltpu.VMEM((2, CH, D), jnp.float32),
            pltpu.VMEM((CH,), jnp.float32),
            pltpu.VMEM((2, CH), jnp.int32),
            pltpu.VMEM((2, CH), jnp.int32),
            pltpu.VMEM((NPAD // 8,), jnp.int32),
            pltpu.VMEM((64,), jnp.float32),
            pltpu.VMEM((NPAD,), jnp.float32),
            pltpu.SemaphoreType.DMA,
            pltpu.SemaphoreType.DMA,
            pltpu.SemaphoreType.DMA,
            pltpu.SemaphoreType.DMA,
            pltpu.SemaphoreType.DMA,
            pltpu.SemaphoreType.DMA,
        ],
    )
    return f(a_pad, b_pad, rcg, vpack, att_tab)


def kernel(h, edge_index, edge_attr, valences, W1, b1, W2, b2,
           U1, bu1, U2, bu2, V1, bv1, V2, bv2):
    vf = valences.astype(jnp.float32)

    x = jnp.zeros((NPAD, KDIM), jnp.float32)
    x = x.at[:N, :D].set(h).at[:N, D].set(vf)
    wa = jnp.zeros((KDIM, D), jnp.float32).at[:D].set(W1[:D]).at[D].set(W1[2 * D])
    wb = jnp.zeros((KDIM, D), jnp.float32).at[:D].set(W1[D:2 * D]).at[D].set(W1[2 * D + 1])

    a_pad, b_pad = _tc_prologue(x, wa, wb, b1[None, :])

    vr_g = jnp.arange(5, dtype=jnp.float32)[:, None] * jnp.ones((1, 5), jnp.float32)
    vc_g = jnp.ones((5, 1), jnp.float32) * jnp.arange(5, dtype=jnp.float32)[None, :]
    pairs = jnp.stack([vr_g.ravel(), vc_g.ravel()], axis=-1)
    tab = jax.nn.sigmoid(jax.nn.relu(pairs @ V1 + bv1) @ V2 + bv2)[:, 0]
    att_tab = jnp.zeros((64,), jnp.float32).at[
        (jnp.arange(25) // 5) * 8 + jnp.arange(25) % 5].set(tab)

    pad_n = EPAD - E
    pad_rows = N + (jnp.arange(pad_n, dtype=jnp.int32) % (NPAD - N))
    row_all = jnp.concatenate([edge_index[0], pad_rows])
    col_all = jnp.concatenate([edge_index[1], pad_rows])
    rcg = (row_all | (col_all << 16)).reshape(NW * CPT, CH)
    rcg = jnp.concatenate([rcg, jnp.zeros((2, CH), jnp.int32)], axis=0)
    v_pad = jnp.zeros((NPAD,), jnp.int32).at[:N].set(valences)
    v8 = v_pad.reshape(NPAD // 8, 8)
    vpack = v8[:, 0]
    for _q in range(1, 8):
        vpack = vpack | (v8[:, _q] << (4 * _q))

    sp, t_out = _sc_edge(a_pad, b_pad, rcg, vpack, att_tab)
    t2 = t_out.reshape(NC * NS, NPAD)

    h_pad = jnp.zeros((NPAD, D), jnp.float32).at[:N].set(h)
    out = _tc_epilogue(h_pad, sp, t2, U1[:D], W2, U1[D:], U2,
                       b2[None, :], bu1[None, :], bu2[None, :])
    return out[:N]


# speedup vs baseline: 24.1709x; 1.3128x over previous
import functools

import jax
import jax.numpy as jnp
from jax import lax
from jax.experimental import pallas as pl
from jax.experimental.pallas import tpu as pltpu
from jax.experimental.pallas import tpu_sc as plsc

N = 10000
E = 320000
D = 128

NC = 2
NS = 16
NW = NC * NS
L = 16

NPAD = 10240
ROWS_PER_TILE = NPAD // NS
CH = 96
CPT = 106
EPT = CH * CPT
EPAD = EPT * NW
KDIM = 136


def _prologue_body(x_ref, wa_ref, wb_ref, b1_ref, a_ref, b_ref):
    x = x_ref[...]
    a_ref[...] = jnp.dot(x, wa_ref[...], preferred_element_type=jnp.float32) + b1_ref[...]
    b_ref[...] = jnp.dot(x, wb_ref[...], preferred_element_type=jnp.float32)


def _tc_prologue(x, wa, wb, b1):
    blk = 1024
    grid = NPAD // blk
    return pl.pallas_call(
        _prologue_body,
        grid=(grid,),
        in_specs=[
            pl.BlockSpec((blk, KDIM), lambda i: (i, 0)),
            pl.BlockSpec((KDIM, D), lambda i: (0, 0)),
            pl.BlockSpec((KDIM, D), lambda i: (0, 0)),
            pl.BlockSpec((1, D), lambda i: (0, 0)),
        ],
        out_specs=[
            pl.BlockSpec((blk, D), lambda i: (i, 0)),
            pl.BlockSpec((blk, D), lambda i: (i, 0)),
        ],
        out_shape=[
            jax.ShapeDtypeStruct((NPAD, D), jnp.float32),
            jax.ShapeDtypeStruct((NPAD, D), jnp.float32),
        ],
    )(x, wa, wb, b1)


def _epilogue_body(h_ref, sp_ref, t_ref, u1a_ref, w2_ref, u1b_ref, u2_ref,
                   b2_ref, bu1_ref, bu2_ref, o_ref):
    hb = h_ref[...]
    s = sp_ref[0] + sp_ref[1]
    w2u = jnp.dot(w2_ref[...], u1b_ref[...], preferred_element_type=jnp.float32)
    b2u = jnp.dot(b2_ref[...], u1b_ref[...], preferred_element_type=jnp.float32)
    ones = jnp.ones((NW, 1), jnp.float32)
    tv = lax.dot_general(t_ref[...], ones, (((0,), (0,)), ((), ())),
                         preferred_element_type=jnp.float32)
    z = (jnp.dot(hb, u1a_ref[...], preferred_element_type=jnp.float32)
         + jnp.dot(s, w2u, preferred_element_type=jnp.float32)
         + tv * b2u
         + bu1_ref[...])
    u = jnp.maximum(z, 0.0)
    o_ref[...] = hb + jnp.dot(u, u2_ref[...], preferred_element_type=jnp.float32) + bu2_ref[...]


def _tc_epilogue(h_pad, sp, t2, u1a, w2, u1b, u2, b2, bu1, bu2):
    blk = 1024
    grid = NPAD // blk
    return pl.pallas_call(
        _epilogue_body,
        grid=(grid,),
        in_specs=[
            pl.BlockSpec((blk, D), lambda i: (i, 0)),
            pl.BlockSpec((2, blk, D), lambda i: (0, i, 0)),
            pl.BlockSpec((NW, blk), lambda i: (0, i)),
            pl.BlockSpec((D, D), lambda i: (0, 0)),
            pl.BlockSpec((D, D), lambda i: (0, 0)),
            pl.BlockSpec((D, D), lambda i: (0, 0)),
            pl.BlockSpec((D, D), lambda i: (0, 0)),
            pl.BlockSpec((1, D), lambda i: (0, 0)),
            pl.BlockSpec((1, D), lambda i: (0, 0)),
            pl.BlockSpec((1, D), lambda i: (0, 0)),
        ],
        out_specs=pl.BlockSpec((blk, D), lambda i: (i, 0)),
        out_shape=jax.ShapeDtypeStruct((NPAD, D), jnp.float32),
    )(h_pad, sp, t2, u1a, w2, u1b, u2, b2, bu1, bu2)


def _sc_edge_body(a_hbm, b_hbm, rcg_hbm, vp_hbm, att_hbm,
                  sp_out, t_out,
                  s_sh, rci, abuf, attb, rowu, colu, vp_loc, att_v,
                  t_loc, sem_a, sem_b, sem_i0, sem_i1, sem_s0, sem_s1):
    cid = lax.axis_index("c")
    sid = lax.axis_index("s")
    wid = sid * NC + cid
    crow = wid * CPT

    pltpu.sync_copy(vp_hbm, vp_loc)
    pltpu.sync_copy(att_hbm, att_v)

    zeros16 = jnp.zeros((L,), jnp.float32)
    iota16 = lax.iota(jnp.int32, L)
    sem_i = (sem_i0, sem_i1)
    sem_s = (sem_s0, sem_s1)

    def _zero_row(r, _):
        for b in range(2):
            for j in range(D // L):
                abuf[b, r, pl.ds(j * L, L)] = zeros16
        return 0

    lax.fori_loop(0, CH, _zero_row, 0)

    def _zero_t(r, _):
        t_loc[pl.ds(r * L, L)] = zeros16
        return 0

    lax.fori_loop(0, NPAD // L, _zero_t, 0)

    base = sid * ROWS_PER_TILE
    for p in range(ROWS_PER_TILE // CH):
        pltpu.sync_copy(abuf.at[0], s_sh.at[pl.ds(base + p * CH, CH)])
    for p in range(ROWS_PER_TILE % CH // L):
        pltpu.sync_copy(abuf.at[0, pl.ds(0, L)],
                        s_sh.at[pl.ds(base + (ROWS_PER_TILE // CH) * CH + p * L, L)])
    plsc.subcore_barrier()

    def _unpack(k, b, r):
        pltpu.make_async_copy(rcg_hbm.at[pl.ds(0, 1)], rci.at[pl.ds(b, 1)],
                              sem_i[b]).wait()
        for g in range(CH // L):
            sl = pl.ds(g * L, L)
            rc16 = rci[b, sl]
            rowu[r, sl] = jnp.bitwise_and(rc16, jnp.int32(0xFFFF))
            colu[b, sl] = jnp.right_shift(rc16, jnp.int32(16))
        pltpu.async_copy(rcg_hbm.at[pl.ds(crow + k + 2, 1)],
                         rci.at[pl.ds(b, 1)], sem_i[b])

    def _att_t(b, r):
        for g in range(CH // L):
            sl = pl.ds(g * L, L)
            r16 = rowu[r, sl]
            c16 = colu[b, sl]
            vr = jnp.bitwise_and(
                jnp.right_shift(plsc.load_gather(vp_loc, [jnp.right_shift(r16, jnp.int32(3))]),
                                jnp.left_shift(jnp.bitwise_and(r16, jnp.int32(7)), jnp.int32(2))),
                jnp.int32(15))
            vc = jnp.bitwise_and(
                jnp.right_shift(plsc.load_gather(vp_loc, [jnp.right_shift(c16, jnp.int32(3))]),
                                jnp.left_shift(jnp.bitwise_and(c16, jnp.int32(7)), jnp.int32(2))),
                jnp.int32(15))
            iv = vr * 8 + vc
            a16 = plsc.load_gather(att_v, [iv])
            attb[b, sl] = a16
            for l in range(L):
                plsc.addupdate_scatter(t_loc, [r16], a16, mask=iota16 == l)

    pltpu.async_copy(rcg_hbm.at[pl.ds(crow, 1)], rci.at[pl.ds(0, 1)], sem_i[0])
    pltpu.async_copy(rcg_hbm.at[pl.ds(crow + 1, 1)], rci.at[pl.ds(1, 1)],
                     sem_i[1])
    for g in range(CH // L):
        rowu[2, pl.ds(g * L, L)] = iota16 + (NPAD - L)
    pltpu.async_copy(abuf.at[1], s_sh.at[rowu.at[2]], sem_s[1], add=True)
    _unpack(0, 0, 0)
    pltpu.async_copy(a_hbm.at[rowu.at[0]], abuf.at[0], sem_a)
    _att_t(0, 0)

    def _do_chunk(k, b):
        b1 = (b + 1) % 2
        r0 = k % 3
        r1 = (k + 1) % 3
        pltpu.make_async_copy(a_hbm.at[rowu.at[r0]], abuf.at[b], sem_a).wait()
        pltpu.async_copy(b_hbm.at[colu.at[b]], abuf.at[b], sem_b, add=True)
        _unpack(k + 1, b1, r1)
        _att_t(b1, r1)
        pltpu.make_async_copy(abuf.at[b1], s_sh.at[rowu.at[r1]],
                              sem_s[b1]).wait()
        pltpu.async_copy(a_hbm.at[rowu.at[r1]], abuf.at[b1], sem_a)
        pltpu.make_async_copy(b_hbm.at[colu.at[b]], abuf.at[b], sem_b).wait()

        def _group(g, _):
            att16 = attb[b, pl.ds(g * L, L)]
            for l in range(L):
                ae = att16[l]
                e = g * L + l
                for j in range(D // L):
                    sl = pl.ds(j * L, L)
                    abuf[b, e, sl] = jnp.maximum(abuf[b, e, sl], 0.0) * ae
            return 0

        lax.fori_loop(0, CH // L, _group, 0)
        pltpu.async_copy(abuf.at[b], s_sh.at[rowu.at[r0]], sem_s[b], add=True)

    def _pair(i, _):
        _do_chunk(i * 2, 0)
        _do_chunk(i * 2 + 1, 1)
        return 0

    lax.fori_loop(0, CPT // 2, _pair, 0)

    pltpu.make_async_copy(abuf.at[1], s_sh.at[rowu.at[0]], sem_s[1]).wait()
    pltpu.make_async_copy(a_hbm.at[rowu.at[0]], abuf.at[0], sem_a).wait()
    for b in range(2):
        pltpu.make_async_copy(rcg_hbm.at[pl.ds(0, 1)], rci.at[pl.ds(b, 1)],
                              sem_i[b]).wait()
    plsc.subcore_barrier()

    pltpu.sync_copy(s_sh.at[pl.ds(base, ROWS_PER_TILE)],
                    sp_out.at[cid, pl.ds(base, ROWS_PER_TILE)])
    pltpu.sync_copy(t_loc, t_out.at[cid, sid])


def _sc_edge(a_pad, b_pad, rcg, vpack, att_tab):
    mesh = plsc.VectorSubcoreMesh(core_axis_name="c", subcore_axis_name="s",
                                  num_cores=NC, num_subcores=NS)
    f = pl.kernel(
        _sc_edge_body,
        out_type=[
            jax.ShapeDtypeStruct((NC, NPAD, D), jnp.float32),
            jax.ShapeDtypeStruct((NC, NS, NPAD), jnp.float32),
        ],
        mesh=mesh,
        compiler_params=pltpu.CompilerParams(needs_layout_passes=False),
        scratch_types=[
            pltpu.VMEM_SHARED((NPAD, D), jnp.float32),
            pltpu.VMEM((2, CH), jnp.int32),
            pltpu.VMEM((2, CH, D), jnp.float32),
            pltpu.VMEM((2, CH), jnp.float32),
            pltpu.VMEM((3, CH), jnp.int32),
            pltpu.VMEM((2, CH), jnp.int32),
            pltpu.VMEM((NPAD // 8,), jnp.int32),
            pltpu.VMEM((64,), jnp.float32),
            pltpu.VMEM((NPAD,), jnp.float32),
            pltpu.SemaphoreType.DMA,
            pltpu.SemaphoreType.DMA,
            pltpu.SemaphoreType.DMA,
            pltpu.SemaphoreType.DMA,
            pltpu.SemaphoreType.DMA,
            pltpu.SemaphoreType.DMA,
        ],
    )
    return f(a_pad, b_pad, rcg, vpack, att_tab)


def kernel(h, edge_index, edge_attr, valences, W1, b1, W2, b2,
           U1, bu1, U2, bu2, V1, bv1, V2, bv2):
    vf = valences.astype(jnp.float32)

    x = jnp.zeros((NPAD, KDIM), jnp.float32)
    x = x.at[:N, :D].set(h).at[:N, D].set(vf)
    wa = jnp.zeros((KDIM, D), jnp.float32).at[:D].set(W1[:D]).at[D].set(W1[2 * D])
    wb = jnp.zeros((KDIM, D), jnp.float32).at[:D].set(W1[D:2 * D]).at[D].set(W1[2 * D + 1])

    a_pad, b_pad = _tc_prologue(x, wa, wb, b1[None, :])

    vr_g = jnp.arange(5, dtype=jnp.float32)[:, None] * jnp.ones((1, 5), jnp.float32)
    vc_g = jnp.ones((5, 1), jnp.float32) * jnp.arange(5, dtype=jnp.float32)[None, :]
    pairs = jnp.stack([vr_g.ravel(), vc_g.ravel()], axis=-1)
    tab = jax.nn.sigmoid(jax.nn.relu(pairs @ V1 + bv1) @ V2 + bv2)[:, 0]
    att_tab = jnp.zeros((64,), jnp.float32).at[
        (jnp.arange(25) // 5) * 8 + jnp.arange(25) % 5].set(tab)

    pad_n = EPAD - E
    pad_rows = N + (jnp.arange(pad_n, dtype=jnp.int32) % (NPAD - N))
    row_all = jnp.concatenate([edge_index[0], pad_rows])
    col_all = jnp.concatenate([edge_index[1], pad_rows])
    rcg = (row_all | (col_all << 16)).reshape(NW * CPT, CH)
    safe = jnp.int32((NPAD - L) | ((NPAD - L) << 16))
    rcg = jnp.concatenate([rcg, jnp.full((4, CH), safe, jnp.int32)], axis=0)
    v_pad = jnp.zeros((NPAD,), jnp.int32).at[:N].set(valences)
    v8 = v_pad.reshape(NPAD // 8, 8)
    vpack = v8[:, 0]
    for _q in range(1, 8):
        vpack = vpack | (v8[:, _q] << (4 * _q))

    sp, t_out = _sc_edge(a_pad, b_pad, rcg, vpack, att_tab)
    t2 = t_out.reshape(NC * NS, NPAD)

    h_pad = jnp.zeros((NPAD, D), jnp.float32).at[:N].set(h)
    out = _tc_epilogue(h_pad, sp, t2, U1[:D], W2, U1[D:], U2,
                       b2[None, :], bu1[None, :], bu2[None, :])
    return out[:N]


# speedup vs baseline: 24.5670x; 1.0164x over previous
import functools

import jax
import jax.numpy as jnp
from jax import lax
from jax.experimental import pallas as pl
from jax.experimental.pallas import tpu as pltpu
from jax.experimental.pallas import tpu_sc as plsc

N = 10000
E = 320000
D = 128

NC = 2
NS = 16
NW = NC * NS
L = 16

NPAD = 10240
ROWS_PER_TILE = NPAD // NS
CH = 96
CPT = 106
EPT = CH * CPT
EPAD = EPT * NW
KDIM = 136


def _prologue_body(x_ref, wa_ref, wb_ref, b1_ref, a_ref, b_ref):
    x = x_ref[...]
    a_ref[...] = jnp.dot(x, wa_ref[...], preferred_element_type=jnp.float32) + b1_ref[...]
    b_ref[...] = jnp.dot(x, wb_ref[...], preferred_element_type=jnp.float32)


def _tc_prologue(x, wa, wb, b1):
    blk = 1024
    grid = NPAD // blk
    return pl.pallas_call(
        _prologue_body,
        grid=(grid,),
        in_specs=[
            pl.BlockSpec((blk, KDIM), lambda i: (i, 0)),
            pl.BlockSpec((KDIM, D), lambda i: (0, 0)),
            pl.BlockSpec((KDIM, D), lambda i: (0, 0)),
            pl.BlockSpec((1, D), lambda i: (0, 0)),
        ],
        out_specs=[
            pl.BlockSpec((blk, D), lambda i: (i, 0)),
            pl.BlockSpec((blk, D), lambda i: (i, 0)),
        ],
        out_shape=[
            jax.ShapeDtypeStruct((NPAD, D), jnp.float32),
            jax.ShapeDtypeStruct((NPAD, D), jnp.float32),
        ],
    )(x, wa, wb, b1)


def _epilogue_body(h_ref, sp_ref, t_ref, u1a_ref, w2_ref, u1b_ref, u2_ref,
                   b2_ref, bu1_ref, bu2_ref, o_ref):
    hb = h_ref[...]
    s = sp_ref[0] + sp_ref[1]
    w2u = jnp.dot(w2_ref[...], u1b_ref[...], preferred_element_type=jnp.float32)
    b2u = jnp.dot(b2_ref[...], u1b_ref[...], preferred_element_type=jnp.float32)
    ones = jnp.ones((NW, 1), jnp.float32)
    tv = jnp.dot(t_ref[...], ones, preferred_element_type=jnp.float32)
    z = (jnp.dot(hb, u1a_ref[...], preferred_element_type=jnp.float32)
         + jnp.dot(s, w2u, preferred_element_type=jnp.float32)
         + tv * b2u
         + bu1_ref[...])
    u = jnp.maximum(z, 0.0)
    o_ref[...] = hb + jnp.dot(u, u2_ref[...], preferred_element_type=jnp.float32) + bu2_ref[...]


def _tc_epilogue(h_in, sp, t2, u1a, w2, u1b, u2, b2, bu1, bu2):
    blk = 1000
    grid = N // blk
    return pl.pallas_call(
        _epilogue_body,
        grid=(grid,),
        in_specs=[
            pl.BlockSpec((blk, D), lambda i: (i, 0)),
            pl.BlockSpec((2, blk, D), lambda i: (0, i, 0)),
            pl.BlockSpec((blk, NW), lambda i: (i, 0)),
            pl.BlockSpec((D, D), lambda i: (0, 0)),
            pl.BlockSpec((D, D), lambda i: (0, 0)),
            pl.BlockSpec((D, D), lambda i: (0, 0)),
            pl.BlockSpec((D, D), lambda i: (0, 0)),
            pl.BlockSpec((1, D), lambda i: (0, 0)),
            pl.BlockSpec((1, D), lambda i: (0, 0)),
            pl.BlockSpec((1, D), lambda i: (0, 0)),
        ],
        out_specs=pl.BlockSpec((blk, D), lambda i: (i, 0)),
        out_shape=jax.ShapeDtypeStruct((N, D), jnp.float32),
    )(h_in, sp, t2, u1a, w2, u1b, u2, b2, bu1, bu2)


def _sc_edge_body(a_hbm, b_hbm, rcg_hbm, vp_hbm, att_hbm,
                  sp_out, t_out,
                  s_sh, rci, abuf, attb, rowu, colu, vp_loc, att_v,
                  t_loc, sem_a, sem_b, sem_i0, sem_i1, sem_s0, sem_s1):
    cid = lax.axis_index("c")
    sid = lax.axis_index("s")
    wid = sid * NC + cid
    crow = wid * CPT

    pltpu.sync_copy(vp_hbm, vp_loc)
    pltpu.sync_copy(att_hbm, att_v)

    zeros16 = jnp.zeros((L,), jnp.float32)
    iota16 = lax.iota(jnp.int32, L)
    sem_i = (sem_i0, sem_i1)
    sem_s = (sem_s0, sem_s1)

    def _zero_row(r, _):
        for b in range(2):
            for j in range(D // L):
                abuf[b, r, pl.ds(j * L, L)] = zeros16
        return 0

    lax.fori_loop(0, CH, _zero_row, 0)

    def _zero_t(r, _):
        t_loc[pl.ds(r * L, L)] = zeros16
        return 0

    lax.fori_loop(0, NPAD // L, _zero_t, 0)

    base = sid * ROWS_PER_TILE
    for p in range(ROWS_PER_TILE // CH):
        pltpu.sync_copy(abuf.at[0], s_sh.at[pl.ds(base + p * CH, CH)])
    for p in range(ROWS_PER_TILE % CH // L):
        pltpu.sync_copy(abuf.at[0, pl.ds(0, L)],
                        s_sh.at[pl.ds(base + (ROWS_PER_TILE // CH) * CH + p * L, L)])
    plsc.subcore_barrier()

    def _unpack(k, b, r):
        pltpu.make_async_copy(rcg_hbm.at[pl.ds(0, 1)], rci.at[pl.ds(b, 1)],
                              sem_i[b]).wait()
        for g in range(CH // L):
            sl = pl.ds(g * L, L)
            rc16 = rci[b, sl]
            rowu[r, sl] = jnp.bitwise_and(rc16, jnp.int32(0xFFFF))
            colu[b, sl] = jnp.right_shift(rc16, jnp.int32(16))
        pltpu.async_copy(rcg_hbm.at[pl.ds(crow + k + 2, 1)],
                         rci.at[pl.ds(b, 1)], sem_i[b])

    def _att_t(b, r):
        for g in range(CH // L):
            sl = pl.ds(g * L, L)
            r16 = rowu[r, sl]
            c16 = colu[b, sl]
            vr = jnp.bitwise_and(
                jnp.right_shift(plsc.load_gather(vp_loc, [jnp.right_shift(r16, jnp.int32(3))]),
                                jnp.left_shift(jnp.bitwise_and(r16, jnp.int32(7)), jnp.int32(2))),
                jnp.int32(15))
            vc = jnp.bitwise_and(
                jnp.right_shift(plsc.load_gather(vp_loc, [jnp.right_shift(c16, jnp.int32(3))]),
                                jnp.left_shift(jnp.bitwise_and(c16, jnp.int32(7)), jnp.int32(2))),
                jnp.int32(15))
            iv = vr * 8 + vc
            a16 = plsc.load_gather(att_v, [iv])
            attb[b, sl] = a16
            for l in range(L):
                plsc.addupdate_scatter(t_loc, [r16], a16, mask=iota16 == l)

    pltpu.async_copy(rcg_hbm.at[pl.ds(crow, 1)], rci.at[pl.ds(0, 1)], sem_i[0])
    pltpu.async_copy(rcg_hbm.at[pl.ds(crow + 1, 1)], rci.at[pl.ds(1, 1)],
                     sem_i[1])
    for g in range(CH // L):
        rowu[2, pl.ds(g * L, L)] = iota16 + (NPAD - L)
    pltpu.async_copy(abuf.at[1], s_sh.at[rowu.at[2]], sem_s[1], add=True)
    _unpack(0, 0, 0)
    pltpu.async_copy(a_hbm.at[rowu.at[0]], abuf.at[0], sem_a)
    _att_t(0, 0)

    def _do_chunk(k, b):
        b1 = (b + 1) % 2
        r0 = k % 3
        r1 = (k + 1) % 3
        pltpu.make_async_copy(a_hbm.at[rowu.at[r0]], abuf.at[b], sem_a).wait()
        pltpu.async_copy(b_hbm.at[colu.at[b]], abuf.at[b], sem_b, add=True)
        _unpack(k + 1, b1, r1)
        _att_t(b1, r1)
        pltpu.make_async_copy(abuf.at[b1], s_sh.at[rowu.at[r1]],
                              sem_s[b1]).wait()
        pltpu.async_copy(a_hbm.at[rowu.at[r1]], abuf.at[b1], sem_a)
        pltpu.make_async_copy(b_hbm.at[colu.at[b]], abuf.at[b], sem_b).wait()

        @plsc.parallel_loop(0, CH // L, 1, unroll=2)
        def _group(g):
            att16 = attb[b, pl.ds(g * L, L)]
            for l in range(L):
                ae = att16[l]
                e = g * L + l
                for j in range(D // L):
                    sl = pl.ds(j * L, L)
                    abuf[b, e, sl] = jnp.maximum(abuf[b, e, sl], 0.0) * ae
        pltpu.async_copy(abuf.at[b], s_sh.at[rowu.at[r0]], sem_s[b], add=True)

    def _pair(i, _):
        _do_chunk(i * 2, 0)
        _do_chunk(i * 2 + 1, 1)
        return 0

    lax.fori_loop(0, CPT // 2, _pair, 0)

    pltpu.make_async_copy(abuf.at[1], s_sh.at[rowu.at[0]], sem_s[1]).wait()
    pltpu.make_async_copy(a_hbm.at[rowu.at[0]], abuf.at[0], sem_a).wait()
    for b in range(2):
        pltpu.make_async_copy(rcg_hbm.at[pl.ds(0, 1)], rci.at[pl.ds(b, 1)],
                              sem_i[b]).wait()
    plsc.subcore_barrier()

    pltpu.sync_copy(s_sh.at[pl.ds(base, ROWS_PER_TILE)],
                    sp_out.at[cid, pl.ds(base, ROWS_PER_TILE)])
    pltpu.sync_copy(t_loc, t_out.at[cid, sid])


def _sc_edge(a_pad, b_pad, rcg, vpack, att_tab):
    mesh = plsc.VectorSubcoreMesh(core_axis_name="c", subcore_axis_name="s",
                                  num_cores=NC, num_subcores=NS)
    f = pl.kernel(
        _sc_edge_body,
        out_type=[
            jax.ShapeDtypeStruct((NC, NPAD, D), jnp.float32),
            jax.ShapeDtypeStruct((NC, NS, NPAD), jnp.float32),
        ],
        mesh=mesh,
        compiler_params=pltpu.CompilerParams(needs_layout_passes=False),
        scratch_types=[
            pltpu.VMEM_SHARED((NPAD, D), jnp.float32),
            pltpu.VMEM((2, CH), jnp.int32),
            pltpu.VMEM((2, CH, D), jnp.float32),
            pltpu.VMEM((2, CH), jnp.float32),
            pltpu.VMEM((3, CH), jnp.int32),
            pltpu.VMEM((2, CH), jnp.int32),
            pltpu.VMEM((NPAD // 8,), jnp.int32),
            pltpu.VMEM((64,), jnp.float32),
            pltpu.VMEM((NPAD,), jnp.float32),
            pltpu.SemaphoreType.DMA,
            pltpu.SemaphoreType.DMA,
            pltpu.SemaphoreType.DMA,
            pltpu.SemaphoreType.DMA,
            pltpu.SemaphoreType.DMA,
            pltpu.SemaphoreType.DMA,
        ],
    )
    return f(a_pad, b_pad, rcg, vpack, att_tab)


def kernel(h, edge_index, edge_attr, valences, W1, b1, W2, b2,
           U1, bu1, U2, bu2, V1, bv1, V2, bv2):
    vf = valences.astype(jnp.float32)

    x = jnp.zeros((NPAD, KDIM), jnp.float32)
    x = x.at[:N, :D].set(h).at[:N, D].set(vf)
    wa = jnp.zeros((KDIM, D), jnp.float32).at[:D].set(W1[:D]).at[D].set(W1[2 * D])
    wb = jnp.zeros((KDIM, D), jnp.float32).at[:D].set(W1[D:2 * D]).at[D].set(W1[2 * D + 1])

    a_pad, b_pad = _tc_prologue(x, wa, wb, b1[None, :])

    vr_g = jnp.arange(5, dtype=jnp.float32)[:, None] * jnp.ones((1, 5), jnp.float32)
    vc_g = jnp.ones((5, 1), jnp.float32) * jnp.arange(5, dtype=jnp.float32)[None, :]
    pairs = jnp.stack([vr_g.ravel(), vc_g.ravel()], axis=-1)
    tab = jax.nn.sigmoid(jax.nn.relu(pairs @ V1 + bv1) @ V2 + bv2)[:, 0]
    att_tab = jnp.zeros((64,), jnp.float32).at[
        (jnp.arange(25) // 5) * 8 + jnp.arange(25) % 5].set(tab)

    pad_n = EPAD - E
    pad_rows = N + (jnp.arange(pad_n, dtype=jnp.int32) % (NPAD - N))
    row_all = jnp.concatenate([edge_index[0], pad_rows])
    col_all = jnp.concatenate([edge_index[1], pad_rows])
    rcg = (row_all | (col_all << 16)).reshape(NW * CPT, CH)
    safe = jnp.int32((NPAD - L) | ((NPAD - L) << 16))
    rcg = jnp.concatenate([rcg, jnp.full((4, CH), safe, jnp.int32)], axis=0)
    v_pad = jnp.zeros((NPAD,), jnp.int32).at[:N].set(valences)
    v8 = v_pad.reshape(NPAD // 8, 8)
    vpack = v8[:, 0]
    for _q in range(1, 8):
        vpack = vpack | (v8[:, _q] << (4 * _q))

    sp, t_out = _sc_edge(a_pad, b_pad, rcg, vpack, att_tab)
    t2 = t_out.reshape(NC * NS, NPAD).T

    return _tc_epilogue(h, sp, t2, U1[:D], W2, U1[D:], U2,
                        b2[None, :], bu1[None, :], bu2[None, :])
